# baseline (speedup 1.0000x reference)
import functools

import jax
import jax.numpy as jnp
from jax import lax
from jax.experimental import pallas as pl
from jax.experimental.pallas import tpu as pltpu
from jax.experimental.pallas import tpu_sc as plsc

Q = 2048
D = 1024
K = 100000
KB = 1024
KPAD = 100352
NKB = KPAD // KB
TOPK = 4
BIG_F = 2.0**30
INF_F = float("inf")


def _lex_min4(av, ai, n_out):
    vs, isel = [], []
    for _ in range(n_out):
        mv = jnp.min(av, axis=1, keepdims=True)
        cand = jnp.where(av == mv, ai, BIG_F)
        mi = jnp.min(cand, axis=1, keepdims=True)
        vs.append(mv)
        isel.append(mi)
        av = jnp.where((av == mv) & (ai == mi), INF_F, av)
    return jnp.concatenate(vs, axis=1), jnp.concatenate(isel, axis=1)


NCAND = 7
MERGE_EVERY = 16


def _block_top7_store(dists, ki, vals_ref, idxs_ref, cv_ref, ci_ref):
    H = KB // 2
    iota = lax.broadcasted_iota(jnp.int32, (Q, H), 1).astype(jnp.float32)
    dL = dists[:, :H]
    dR = dists[:, H:]
    takeL = dL <= dR
    v = jnp.where(takeL, dL, dR)
    i = jnp.where(takeL, iota, iota + jnp.float32(H))
    mx = jnp.where(takeL, dR, dL)

    bvs, bis = [], []
    for t in range(TOPK):
        mv = jnp.min(v, axis=1, keepdims=True)
        mi = jnp.min(jnp.where(v == mv, i, BIG_F), axis=1, keepdims=True)
        bvs.append(mv)
        bis.append(mi)
        if t + 1 < TOPK:
            v = jnp.where(i == mi, INF_F, v)
    for t in range(TOPK - 1):
        pmask = i == bis[t]
        bvs.append(jnp.min(jnp.where(pmask, mx, INF_F), axis=1, keepdims=True))
        bis.append(jnp.where(bis[t] < jnp.float32(H),
                             bis[t] + jnp.float32(H), bis[t] - jnp.float32(H)))
    bv = jnp.concatenate(bvs, axis=1)
    bi = jnp.concatenate(bis, axis=1) + jnp.float32(KB) * ki.astype(jnp.float32)

    slot = lax.rem(ki, MERGE_EVERY)
    for j in range(MERGE_EVERY):
        @pl.when(slot == j)
        def _store(j=j):
            cv_ref[:, j * NCAND:(j + 1) * NCAND] = bv
            ci_ref[:, j * NCAND:(j + 1) * NCAND] = bi

    @pl.when((slot == MERGE_EVERY - 1) | (ki == NKB - 1))
    def _merge():
        av = jnp.concatenate([vals_ref[...], cv_ref[...]], axis=1)
        ai = jnp.concatenate([idxs_ref[...], ci_ref[...]], axis=1)
        nv, ni = _lex_min4(av, ai, TOPK)
        vals_ref[...] = nv
        idxs_ref[...] = ni
        cv_ref[...] = jnp.full((Q, MERGE_EVERY * NCAND), jnp.inf, jnp.float32)
        ci_ref[...] = jnp.full((Q, MERGE_EVERY * NCAND), BIG_F, jnp.float32)


def _topk_body(qn2x_ref, q2_ref, m_ref, mn_ref, idx_out_ref,
               vals_ref, idxs_ref, cv_ref, ci_ref):
    ki = pl.program_id(0)

    @pl.when(ki == 0)
    def _init():
        vals_ref[...] = jnp.full((Q, TOPK), jnp.inf, jnp.float32)
        idxs_ref[...] = jnp.full((Q, TOPK), BIG_F, jnp.float32)
        cv_ref[...] = jnp.full((Q, MERGE_EVERY * NCAND), jnp.inf, jnp.float32)
        ci_ref[...] = jnp.full((Q, MERGE_EVERY * NCAND), BIG_F, jnp.float32)

    qn2x = qn2x_ref[...]
    mn = mn_ref[...]

    dot2 = lax.dot_general(q2_ref[...], m_ref[...], (((1,), (1,)), ((), ())),
                           preferred_element_type=jnp.float32)
    qn2 = (qn2x * qn2x) * 0.25
    mn2 = mn * mn
    sq = (qn2 + mn2) - dot2
    dotp = (qn2 - sq) + mn2
    dists = 1.0 - dotp / (qn2x * mn)

    @pl.when(ki < NKB - 1)
    def _mid():
        _block_top7_store(dists, ki, vals_ref, idxs_ref, cv_ref, ci_ref)

    @pl.when(ki == NKB - 1)
    def _last():
        iota = lax.broadcasted_iota(jnp.int32, (Q, KB), 1).astype(jnp.float32)
        masked = jnp.where(iota < jnp.float32(K - (NKB - 1) * KB), dists, INF_F)
        _block_top7_store(masked, ki, vals_ref, idxs_ref, cv_ref, ci_ref)
        idx_out_ref[...] = idxs_ref[...].astype(jnp.int32)


def _topk_call(qn2x, q2, m_pad, mn_pad):
    return pl.pallas_call(
        _topk_body,
        grid=(NKB,),
        in_specs=[
            pl.BlockSpec((Q, 1), lambda k: (0, 0)),
            pl.BlockSpec((Q, D), lambda k: (0, 0)),
            pl.BlockSpec((KB, D), lambda k: (k, 0)),
            pl.BlockSpec((1, KB), lambda k: (0, k)),
        ],
        out_specs=pl.BlockSpec((Q, TOPK), lambda k: (0, 0)),
        out_shape=jax.ShapeDtypeStruct((Q, TOPK), jnp.int32),
        scratch_shapes=[
            pltpu.VMEM((Q, TOPK), jnp.float32),
            pltpu.VMEM((Q, TOPK), jnp.float32),
            pltpu.VMEM((Q, MERGE_EVERY * NCAND), jnp.float32),
            pltpu.VMEM((Q, MERGE_EVERY * NCAND), jnp.float32),
        ],
        compiler_params=pltpu.CompilerParams(
            dimension_semantics=("arbitrary",),
        ),
    )(qn2x, q2, m_pad, mn_pad)


NC = 2
NS = 16
NW = NC * NS
QPW = Q // NW
CH = 16
NCH = QPW // CH
LANES = 16
DBLK = D // LANES


def _gather_mean_body(idx_hbm, synth_hbm, out_hbm, idx_v, rows_v, acc_v, sem):
    wid = lax.axis_index("s") * NC + lax.axis_index("c")
    base = wid * QPW

    def chunk_body(c, carry):
        qbase = base + c * CH
        pltpu.sync_copy(idx_hbm.at[pl.ds(qbase * TOPK, CH * TOPK)], idx_v)
        pltpu.async_copy(synth_hbm.at[idx_v], rows_v, sem).wait()

        def q_body(i, carry2):
            def d_body(j, carry3):
                s0 = rows_v[i * TOPK + 0, pl.ds(j * LANES, LANES)]
                s1 = rows_v[i * TOPK + 1, pl.ds(j * LANES, LANES)]
                s2 = rows_v[i * TOPK + 2, pl.ds(j * LANES, LANES)]
                s3 = rows_v[i * TOPK + 3, pl.ds(j * LANES, LANES)]
                acc_v[i, pl.ds(j * LANES, LANES)] = (
                    ((s0 + s1) + s2) + s3) * jnp.float32(0.25)
                return carry3

            return lax.fori_loop(0, DBLK, d_body, carry2)

        lax.fori_loop(0, CH, q_body, carry)
        pltpu.sync_copy(acc_v, out_hbm.at[pl.ds(qbase, CH)])
        return carry

    lax.fori_loop(0, NCH, chunk_body, 0)


def _gather_mean_call(idx_flat, synth_set):
    mesh = plsc.VectorSubcoreMesh(core_axis_name="c", subcore_axis_name="s")
    kern = functools.partial(
        pl.kernel,
        mesh=mesh,
        out_type=jax.ShapeDtypeStruct((Q, D), jnp.float32),
        scratch_types=[
            pltpu.VMEM((CH * TOPK,), jnp.int32),
            pltpu.VMEM((CH * TOPK, D), jnp.float32),
            pltpu.VMEM((CH, D), jnp.float32),
            pltpu.SemaphoreType.DMA,
        ],
    )(_gather_mean_body)
    return kern(idx_flat, synth_set)


def kernel(query_seq, matching_set, synth_set, topk):
    del topk
    qn = jnp.linalg.norm(query_seq, ord=2, axis=-1)
    mn = jnp.linalg.norm(matching_set, ord=2, axis=-1)
    m_pad = jnp.concatenate(
        [matching_set, jnp.zeros((KPAD - K, D), jnp.float32)], axis=0)
    mn_pad = jnp.concatenate([mn, jnp.ones((KPAD - K,), jnp.float32)])
    qn2x = qn + qn
    q2 = query_seq + query_seq
    idx = _topk_call(qn2x.reshape(Q, 1), q2, m_pad,
                     mn_pad.reshape(1, KPAD))
    return _gather_mean_call(idx.reshape(Q * TOPK), synth_set)


# speedup vs baseline: 1.9716x; 1.9716x over previous
import functools

import jax
import jax.numpy as jnp
from jax import lax
from jax.experimental import pallas as pl
from jax.experimental.pallas import tpu as pltpu
from jax.experimental.pallas import tpu_sc as plsc

Q = 2048
D = 1024
K = 100000
KB = 1024
KPAD = 100352
NKB = KPAD // KB
TOPK = 4
BIG_F = 2.0**30
INF_F = float("inf")


def _lex_min4(av, ai, n_out):
    vs, isel = [], []
    for _ in range(n_out):
        mv = jnp.min(av, axis=1, keepdims=True)
        cand = jnp.where(av == mv, ai, BIG_F)
        mi = jnp.min(cand, axis=1, keepdims=True)
        vs.append(mv)
        isel.append(mi)
        av = jnp.where((av == mv) & (ai == mi), INF_F, av)
    return jnp.concatenate(vs, axis=1), jnp.concatenate(isel, axis=1)


NCAND = 7
MERGE_EVERY = 8


def _block_top7_store(dists, ki, vals_ref, idxs_ref, cv_ref, ci_ref):
    H = KB // 2
    iota = lax.broadcasted_iota(jnp.int32, (Q, H), 1).astype(jnp.float32)
    dL = dists[:, :H]
    dR = dists[:, H:]
    takeL = dL <= dR
    v = jnp.where(takeL, dL, dR)
    i = jnp.where(takeL, iota, iota + jnp.float32(H))
    mx = jnp.where(takeL, dR, dL)

    bvs, bis = [], []
    for t in range(TOPK):
        mv = jnp.min(v, axis=1, keepdims=True)
        mi = jnp.min(jnp.where(v == mv, i, BIG_F), axis=1, keepdims=True)
        bvs.append(mv)
        bis.append(mi)
        if t + 1 < TOPK:
            v = jnp.where(i == mi, INF_F, v)
    for t in range(TOPK - 1):
        pmask = i == bis[t]
        bvs.append(jnp.min(jnp.where(pmask, mx, INF_F), axis=1, keepdims=True))
        bis.append(jnp.where(bis[t] < jnp.float32(H),
                             bis[t] + jnp.float32(H), bis[t] - jnp.float32(H)))
    bv = jnp.concatenate(bvs, axis=1)
    bi = jnp.concatenate(bis, axis=1) + jnp.float32(KB) * ki.astype(jnp.float32)

    slot = lax.rem(ki, MERGE_EVERY)
    for j in range(MERGE_EVERY):
        @pl.when(slot == j)
        def _store(j=j):
            cv_ref[:, j * NCAND:(j + 1) * NCAND] = bv
            ci_ref[:, j * NCAND:(j + 1) * NCAND] = bi

    @pl.when((slot == MERGE_EVERY - 1) | (ki == NKB - 1))
    def _merge():
        av = jnp.concatenate([vals_ref[...], cv_ref[...]], axis=1)
        ai = jnp.concatenate([idxs_ref[...], ci_ref[...]], axis=1)
        nv, ni = _lex_min4(av, ai, TOPK)
        vals_ref[...] = nv
        idxs_ref[...] = ni
        cv_ref[...] = jnp.full((Q, MERGE_EVERY * NCAND), jnp.inf, jnp.float32)
        ci_ref[...] = jnp.full((Q, MERGE_EVERY * NCAND), BIG_F, jnp.float32)


def _topk_body(qn2x_ref, q2_ref, m_ref, mn_ref, idx_out_ref,
               vals_ref, idxs_ref, cv_ref, ci_ref):
    ki = pl.program_id(0)

    @pl.when(ki == 0)
    def _init():
        vals_ref[...] = jnp.full((Q, TOPK), jnp.inf, jnp.float32)
        idxs_ref[...] = jnp.full((Q, TOPK), BIG_F, jnp.float32)
        cv_ref[...] = jnp.full((Q, MERGE_EVERY * NCAND), jnp.inf, jnp.float32)
        ci_ref[...] = jnp.full((Q, MERGE_EVERY * NCAND), BIG_F, jnp.float32)

    qn2x = qn2x_ref[...]
    mn = mn_ref[...]

    dot2 = lax.dot_general(q2_ref[...], m_ref[...], (((1,), (1,)), ((), ())),
                           preferred_element_type=jnp.float32)
    qn2 = (qn2x * qn2x) * 0.25
    mn2 = mn * mn
    sq = (qn2 + mn2) - dot2
    dotp = (qn2 - sq) + mn2
    dists = 1.0 - dotp / (qn2x * mn)

    @pl.when(ki < NKB - 1)
    def _mid():
        _block_top7_store(dists, ki, vals_ref, idxs_ref, cv_ref, ci_ref)

    @pl.when(ki == NKB - 1)
    def _last():
        iota = lax.broadcasted_iota(jnp.int32, (Q, KB), 1).astype(jnp.float32)
        masked = jnp.where(iota < jnp.float32(K - (NKB - 1) * KB), dists, INF_F)
        _block_top7_store(masked, ki, vals_ref, idxs_ref, cv_ref, ci_ref)
        idx_out_ref[...] = idxs_ref[...].astype(jnp.int32)


def _topk_call(qn2x, q2, m_pad, mn_pad):
    return pl.pallas_call(
        _topk_body,
        grid=(NKB,),
        in_specs=[
            pl.BlockSpec((Q, 1), lambda k: (0, 0)),
            pl.BlockSpec((Q, D), lambda k: (0, 0)),
            pl.BlockSpec((KB, D), lambda k: (k, 0)),
            pl.BlockSpec((1, KB), lambda k: (0, k)),
        ],
        out_specs=pl.BlockSpec((Q, TOPK), lambda k: (0, 0)),
        out_shape=jax.ShapeDtypeStruct((Q, TOPK), jnp.int32),
        scratch_shapes=[
            pltpu.VMEM((Q, TOPK), jnp.float32),
            pltpu.VMEM((Q, TOPK), jnp.float32),
            pltpu.VMEM((Q, MERGE_EVERY * NCAND), jnp.float32),
            pltpu.VMEM((Q, MERGE_EVERY * NCAND), jnp.float32),
        ],
        compiler_params=pltpu.CompilerParams(
            dimension_semantics=("arbitrary",),
        ),
    )(qn2x, q2, m_pad, mn_pad)


NC = 2
NS = 16
NW = NC * NS
QPW = Q // NW
CH = 16
NCH = QPW // CH
LANES = 16
DBLK = D // LANES


def _gather_mean_body(idx_hbm, synth_hbm, out_hbm, idx_v, rows_v, acc_v, sem):
    wid = lax.axis_index("s") * NC + lax.axis_index("c")
    base = wid * QPW

    def chunk_body(c, carry):
        qbase = base + c * CH
        pltpu.sync_copy(idx_hbm.at[pl.ds(qbase * TOPK, CH * TOPK)], idx_v)
        pltpu.async_copy(synth_hbm.at[idx_v], rows_v, sem).wait()

        def q_body(i, carry2):
            def d_body(j, carry3):
                s0 = rows_v[i * TOPK + 0, pl.ds(j * LANES, LANES)]
                s1 = rows_v[i * TOPK + 1, pl.ds(j * LANES, LANES)]
                s2 = rows_v[i * TOPK + 2, pl.ds(j * LANES, LANES)]
                s3 = rows_v[i * TOPK + 3, pl.ds(j * LANES, LANES)]
                acc_v[i, pl.ds(j * LANES, LANES)] = (
                    ((s0 + s1) + s2) + s3) * jnp.float32(0.25)
                return carry3

            return lax.fori_loop(0, DBLK, d_body, carry2)

        lax.fori_loop(0, CH, q_body, carry)
        pltpu.sync_copy(acc_v, out_hbm.at[pl.ds(qbase, CH)])
        return carry

    lax.fori_loop(0, NCH, chunk_body, 0)


def _gather_mean_call(idx_flat, synth_set):
    mesh = plsc.VectorSubcoreMesh(core_axis_name="c", subcore_axis_name="s")
    kern = functools.partial(
        pl.kernel,
        mesh=mesh,
        out_type=jax.ShapeDtypeStruct((Q, D), jnp.float32),
        scratch_types=[
            pltpu.VMEM((CH * TOPK,), jnp.int32),
            pltpu.VMEM((CH * TOPK, D), jnp.float32),
            pltpu.VMEM((CH, D), jnp.float32),
            pltpu.SemaphoreType.DMA,
        ],
    )(_gather_mean_body)
    return kern(idx_flat, synth_set)


def kernel(query_seq, matching_set, synth_set, topk):
    del topk
    qn = jnp.linalg.norm(query_seq, ord=2, axis=-1)
    mn = jnp.linalg.norm(matching_set, ord=2, axis=-1)
    m_pad = jnp.concatenate(
        [matching_set, jnp.zeros((KPAD - K, D), jnp.float32)], axis=0)
    mn_pad = jnp.concatenate([mn, jnp.ones((KPAD - K,), jnp.float32)])
    qn2x = qn + qn
    q2 = query_seq + query_seq
    idx = _topk_call(qn2x.reshape(Q, 1), q2, m_pad,
                     mn_pad.reshape(1, KPAD))
    return _gather_mean_call(idx.reshape(Q * TOPK), synth_set)


# speedup vs baseline: 2.3030x; 1.1681x over previous
import functools

import jax
import jax.numpy as jnp
from jax import lax
from jax.experimental import pallas as pl
from jax.experimental.pallas import tpu as pltpu
from jax.experimental.pallas import tpu_sc as plsc

Q = 2048
D = 1024
K = 100000
KB = 1024
NKB = -(-K // KB)
TOPK = 4
BIG_F = 2.0**30
INF_F = float("inf")


def _lex_min4(av, ai, n_out):
    vs, isel = [], []
    for _ in range(n_out):
        mv = jnp.min(av, axis=1, keepdims=True)
        cand = jnp.where(av == mv, ai, BIG_F)
        mi = jnp.min(cand, axis=1, keepdims=True)
        vs.append(mv)
        isel.append(mi)
        av = jnp.where((av == mv) & (ai == mi), INF_F, av)
    return jnp.concatenate(vs, axis=1), jnp.concatenate(isel, axis=1)


NCAND = 7
MERGE_EVERY = 8


def _block_top7_store(dists, ki, vals_ref, idxs_ref, cv_ref, ci_ref):
    H = KB // 2
    iota = lax.broadcasted_iota(jnp.int32, (Q, H), 1).astype(jnp.float32)
    dL = dists[:, :H]
    dR = dists[:, H:]
    takeL = dL <= dR
    v = jnp.where(takeL, dL, dR)
    i = jnp.where(takeL, iota, iota + jnp.float32(H))
    mx = jnp.where(takeL, dR, dL)

    bvs, bis = [], []
    for t in range(TOPK):
        mv = jnp.min(v, axis=1, keepdims=True)
        mi = jnp.min(jnp.where(v == mv, i, BIG_F), axis=1, keepdims=True)
        bvs.append(mv)
        bis.append(mi)
        if t + 1 < TOPK:
            v = jnp.where(i == mi, INF_F, v)
    for t in range(TOPK - 1):
        pmask = i == bis[t]
        bvs.append(jnp.min(jnp.where(pmask, mx, INF_F), axis=1, keepdims=True))
        bis.append(jnp.where(bis[t] < jnp.float32(H),
                             bis[t] + jnp.float32(H), bis[t] - jnp.float32(H)))
    bv = jnp.concatenate(bvs, axis=1)
    bi = jnp.concatenate(bis, axis=1) + jnp.float32(KB) * ki.astype(jnp.float32)

    slot = lax.rem(ki, MERGE_EVERY)
    for j in range(MERGE_EVERY):
        @pl.when(slot == j)
        def _store(j=j):
            cv_ref[:, j * NCAND:(j + 1) * NCAND] = bv
            ci_ref[:, j * NCAND:(j + 1) * NCAND] = bi

    @pl.when((slot == MERGE_EVERY - 1) | (ki == NKB - 1))
    def _merge():
        av = jnp.concatenate([vals_ref[...], cv_ref[...]], axis=1)
        ai = jnp.concatenate([idxs_ref[...], ci_ref[...]], axis=1)
        nv, ni = _lex_min4(av, ai, TOPK)
        vals_ref[...] = nv
        idxs_ref[...] = ni
        cv_ref[...] = jnp.full((Q, MERGE_EVERY * NCAND), jnp.inf, jnp.float32)
        ci_ref[...] = jnp.full((Q, MERGE_EVERY * NCAND), BIG_F, jnp.float32)


def _topk_body(qn2x_ref, q2_ref, m_ref, mn_ref, idx_out_ref,
               vals_ref, idxs_ref, cv_ref, ci_ref):
    ki = pl.program_id(0)

    @pl.when(ki == 0)
    def _init():
        vals_ref[...] = jnp.full((Q, TOPK), jnp.inf, jnp.float32)
        idxs_ref[...] = jnp.full((Q, TOPK), BIG_F, jnp.float32)
        cv_ref[...] = jnp.full((Q, MERGE_EVERY * NCAND), jnp.inf, jnp.float32)
        ci_ref[...] = jnp.full((Q, MERGE_EVERY * NCAND), BIG_F, jnp.float32)

    qn2x = qn2x_ref[...]
    mn = mn_ref[...]

    dot2 = lax.dot_general(q2_ref[...], m_ref[...], (((1,), (1,)), ((), ())),
                           preferred_element_type=jnp.float32)
    qn2 = (qn2x * qn2x) * 0.25
    mn2 = mn * mn
    sq = (qn2 + mn2) - dot2
    dotp = (qn2 - sq) + mn2
    dists = 1.0 - dotp / (qn2x * mn)

    @pl.when(ki < NKB - 1)
    def _mid():
        _block_top7_store(dists, ki, vals_ref, idxs_ref, cv_ref, ci_ref)

    @pl.when(ki == NKB - 1)
    def _last():
        iota = lax.broadcasted_iota(jnp.int32, (Q, KB), 1).astype(jnp.float32)
        masked = jnp.where(iota < jnp.float32(K - (NKB - 1) * KB), dists, INF_F)
        _block_top7_store(masked, ki, vals_ref, idxs_ref, cv_ref, ci_ref)
        idx_out_ref[...] = idxs_ref[...].astype(jnp.int32)


def _topk_call(qn2x, q2, m_pad, mn_pad):
    return pl.pallas_call(
        _topk_body,
        grid=(NKB,),
        in_specs=[
            pl.BlockSpec((Q, 1), lambda k: (0, 0)),
            pl.BlockSpec((Q, D), lambda k: (0, 0)),
            pl.BlockSpec((KB, D), lambda k: (k, 0)),
            pl.BlockSpec((1, KB), lambda k: (0, k)),
        ],
        out_specs=pl.BlockSpec((Q, TOPK), lambda k: (0, 0)),
        out_shape=jax.ShapeDtypeStruct((Q, TOPK), jnp.int32),
        scratch_shapes=[
            pltpu.VMEM((Q, TOPK), jnp.float32),
            pltpu.VMEM((Q, TOPK), jnp.float32),
            pltpu.VMEM((Q, MERGE_EVERY * NCAND), jnp.float32),
            pltpu.VMEM((Q, MERGE_EVERY * NCAND), jnp.float32),
        ],
        compiler_params=pltpu.CompilerParams(
            dimension_semantics=("arbitrary",),
        ),
    )(qn2x, q2, m_pad, mn_pad)


NC = 2
NS = 16
NW = NC * NS
QPW = Q // NW
CH = 16
NCH = QPW // CH
LANES = 16
DBLK = D // LANES


def _gather_mean_body(idx_hbm, synth_hbm, out_hbm, idx_v, rows_v, acc_v, sem):
    wid = lax.axis_index("s") * NC + lax.axis_index("c")
    base = wid * QPW

    def chunk_body(c, carry):
        qbase = base + c * CH
        pltpu.sync_copy(idx_hbm.at[pl.ds(qbase * TOPK, CH * TOPK)], idx_v)
        pltpu.async_copy(synth_hbm.at[idx_v], rows_v, sem).wait()

        def q_body(i, carry2):
            def d_body(j, carry3):
                s0 = rows_v[i * TOPK + 0, pl.ds(j * LANES, LANES)]
                s1 = rows_v[i * TOPK + 1, pl.ds(j * LANES, LANES)]
                s2 = rows_v[i * TOPK + 2, pl.ds(j * LANES, LANES)]
                s3 = rows_v[i * TOPK + 3, pl.ds(j * LANES, LANES)]
                acc_v[i, pl.ds(j * LANES, LANES)] = (
                    ((s0 + s1) + s2) + s3) * jnp.float32(0.25)
                return carry3

            return lax.fori_loop(0, DBLK, d_body, carry2)

        lax.fori_loop(0, CH, q_body, carry)
        pltpu.sync_copy(acc_v, out_hbm.at[pl.ds(qbase, CH)])
        return carry

    lax.fori_loop(0, NCH, chunk_body, 0)


def _gather_mean_call(idx_flat, synth_set):
    mesh = plsc.VectorSubcoreMesh(core_axis_name="c", subcore_axis_name="s")
    kern = functools.partial(
        pl.kernel,
        mesh=mesh,
        out_type=jax.ShapeDtypeStruct((Q, D), jnp.float32),
        scratch_types=[
            pltpu.VMEM((CH * TOPK,), jnp.int32),
            pltpu.VMEM((CH * TOPK, D), jnp.float32),
            pltpu.VMEM((CH, D), jnp.float32),
            pltpu.SemaphoreType.DMA,
        ],
    )(_gather_mean_body)
    return kern(idx_flat, synth_set)


def kernel(query_seq, matching_set, synth_set, topk):
    del topk
    qn = jnp.linalg.norm(query_seq, ord=2, axis=-1)
    mn = jnp.linalg.norm(matching_set, ord=2, axis=-1)
    qn2x = qn + qn
    q2 = query_seq + query_seq
    idx = _topk_call(qn2x.reshape(Q, 1), q2, matching_set, mn.reshape(1, K))
    return _gather_mean_call(idx.reshape(Q * TOPK), synth_set)
